# baseline (speedup 1.0000x reference)
import functools

import jax
import jax.numpy as jnp
import numpy as np
from jax import lax
from jax.experimental import pallas as pl
from jax.experimental.pallas import tpu as pltpu
from jax.experimental.pallas import tpu_sc as plsc

B, N, T, H = 16, 256, 128, 16
LANES = 16
SC_H = 8
H_PER = 4
CHUNK = 16
NCHUNK = N // CHUNK
B_GRP = 1
NITEMS = B_GRP * NCHUNK
NJB = N // LANES

_FMAX = np.float32(np.finfo(np.float32).max)
_FMIN = np.float32(np.finfo(np.float32).min)

_TAKE_DNUMS = lax.GatherDimensionNumbers(
    offset_dims=(), collapsed_slice_dims=(0,), start_index_map=(0,)
)


def _lane_splat(vec, lane):
    idx = jnp.broadcast_to(lane, (LANES,)).astype(jnp.int32)
    return lax.gather(
        vec,
        idx[:, None],
        dimension_numbers=_TAKE_DNUMS,
        slice_sizes=(1,),
        mode=lax.GatherScatterMode.PROMISE_IN_BOUNDS,
    )


def _logd_pass(d, p_table):
    def body(d_ref, pt_ref, o_ref, ptt_ref):
        x = jnp.log(d_ref[...])
        x = jnp.where(jnp.isnan(x), _FMAX, x)
        o_ref[...] = jnp.clip(x, _FMIN, _FMAX)

        @pl.when(pl.program_id(0) == 0)
        def _():
            ptt_ref[...] = jnp.transpose(pt_ref[...], (2, 0, 1))

    return pl.pallas_call(
        body,
        grid=(d.shape[0],),
        in_specs=[
            pl.BlockSpec((1, N, N), lambda b: (b, 0, 0)),
            pl.BlockSpec((T, T, H), lambda b: (0, 0, 0)),
        ],
        out_specs=[
            pl.BlockSpec((1, N, N), lambda b: (b, 0, 0)),
            pl.BlockSpec((H, T, T), lambda b: (0, 0, 0)),
        ],
        out_shape=[
            jax.ShapeDtypeStruct(d.shape, jnp.float32),
            jax.ShapeDtypeStruct((H, T, T), jnp.float32),
        ],
    )(d, p_table)


_mesh = plsc.VectorSubcoreMesh(core_axis_name="c", subcore_axis_name="s")


@functools.partial(
    pl.kernel,
    mesh=_mesh,
    out_type=jax.ShapeDtypeStruct((B, H, N, N), jnp.float32),
    scratch_types=[
        pltpu.VMEM((T * T,), jnp.float32),
        pltpu.VMEM((T * T,), jnp.float32),
        pltpu.VMEM((T * T,), jnp.float32),
        pltpu.VMEM((T * T,), jnp.float32),
        pltpu.VMEM((B_GRP, N), jnp.int32),
        pltpu.VMEM((CHUNK, N), jnp.float32),
        pltpu.VMEM((CHUNK, N), jnp.float32),
        pltpu.VMEM((H_PER, CHUNK, N), jnp.float32),
        pltpu.VMEM((H_PER, CHUNK, N), jnp.float32),
        pltpu.SemaphoreType.DMA,
        pltpu.SemaphoreType.DMA,
        pltpu.SemaphoreType.DMA,
        pltpu.SemaphoreType.DMA,
        pltpu.SemaphoreType.DMA,
    ],
    compiler_params=pltpu.CompilerParams(needs_layout_passes=False),
)
def _sc_pass(logd_hbm, tok_hbm, pt_hbm, out_hbm,
             p_h0, p_h1, p_h2, p_h3, tok_v, ld0, ld1, ob0, ob1,
             is0, is1, os0, os1, psem):
    c = lax.axis_index("c")
    s = lax.axis_index("s")
    h0 = (s % 2) * H_PER
    bg = c * 8 + s // 2
    ld = (ld0, ld1)
    ob = (ob0, ob1)
    isem = (is0, is1)
    osem = (os0, os1)
    planes = (p_h0, p_h1, p_h2, p_h3)

    def item_bcc(g):
        lb = g // NCHUNK
        return lb, bg * B_GRP + lb, g % NCHUNK

    def start_in(g, par):
        _, b, cc = item_bcc(g)
        pltpu.make_async_copy(
            logd_hbm.at[b, pl.ds(cc * CHUNK, CHUNK)], ld[par], isem[par]
        ).start()

    plane_copies = [
        pltpu.make_async_copy(pt_hbm.at[h0 + k], planes[k], psem)
        for k in range(H_PER)
    ]
    tok_copy = pltpu.make_async_copy(
        tok_hbm.at[pl.ds(bg * B_GRP, B_GRP)], tok_v, psem
    )
    for cp in plane_copies:
        cp.start()
    tok_copy.start()
    start_in(0, 0)
    for cp in plane_copies:
        cp.wait()
    tok_copy.wait()

    def pair_body(k, carry):
        for par in (0, 1):
            g = k * 2 + par
            lb, b, cc = item_bcc(g)

            @pl.when(g + 1 < NITEMS)
            def _():
                start_in(g + 1, 1 - par)

            pltpu.make_async_copy(
                logd_hbm.at[b, pl.ds(cc * CHUNK, CHUNK)], ld[par], isem[par]
            ).wait()

            @pl.when(g >= 2)
            def _():
                pltpu.make_async_copy(
                    ob[par],
                    out_hbm.at[b, pl.ds(h0, H_PER), pl.ds(cc * CHUNK, CHUNK)],
                    osem[par],
                ).wait()

            tjs = [tok_v[lb, pl.ds(jb * LANES, LANES)] for jb in range(NJB)]
            ldb = ld[par]
            obb = ob[par]
            tiv = tok_v[lb, pl.ds(cc * CHUNK, CHUNK)]

            @plsc.parallel_loop(0, CHUNK)
            def i_loop(r, tiv=tiv, ldb=ldb, obb=obb, tjs=tjs):
                base = _lane_splat(tiv, r) * T
                for jb in range(NJB):
                    idx = base + tjs[jb]
                    lvec = ldb[r, pl.ds(jb * LANES, LANES)]
                    g0 = plsc.load_gather(p_h0, [idx])
                    g1 = plsc.load_gather(p_h1, [idx])
                    g2 = plsc.load_gather(p_h2, [idx])
                    g3 = plsc.load_gather(p_h3, [idx])
                    obb[0, r, pl.ds(jb * LANES, LANES)] = g0 * lvec
                    obb[1, r, pl.ds(jb * LANES, LANES)] = g1 * lvec
                    obb[2, r, pl.ds(jb * LANES, LANES)] = g2 * lvec
                    obb[3, r, pl.ds(jb * LANES, LANES)] = g3 * lvec

            pltpu.make_async_copy(
                obb,
                out_hbm.at[b, pl.ds(h0, H_PER), pl.ds(cc * CHUNK, CHUNK)],
                osem[par],
            ).start()
        return carry

    lax.fori_loop(0, NITEMS // 2, pair_body, 0)

    for par in (0, 1):
        g = NITEMS - 2 + par
        _, b, cc = item_bcc(g)
        pltpu.make_async_copy(
            ob[par],
            out_hbm.at[b, pl.ds(h0, H_PER), pl.ds(cc * CHUNK, CHUNK)],
            osem[par],
        ).wait()


def _mm_pass(out_sc, logd, tok, pt_hi):
    n_hi = H - SC_H

    def body(osc_ref, ld_ref, tok_ref, pth_ref, o_ref, oh_ref):
        hh = pl.program_id(1)

        @pl.when(hh == 0)
        def _():
            t = tok_ref[pl.program_id(0)]
            iot = lax.broadcasted_iota(jnp.int32, (N, T), 1)
            oh_ref[...] = (t[:, None] == iot).astype(jnp.float32)

        oh = oh_ref[...]
        p_h = pth_ref[hh]
        a = jnp.dot(oh, p_h, preferred_element_type=jnp.float32)
        out = lax.dot_general(
            a, oh, (((1,), (1,)), ((), ())),
            preferred_element_type=jnp.float32,
        )
        o_ref[0, 0] = out * ld_ref[0]

    return pl.pallas_call(
        body,
        grid=(B, n_hi),
        in_specs=[
            pl.BlockSpec(memory_space=pl.ANY),
            pl.BlockSpec((1, N, N), lambda b, hh: (b, 0, 0)),
            pl.BlockSpec((B, N), lambda b, hh: (0, 0)),
            pl.BlockSpec((n_hi, T, T), lambda b, hh: (0, 0, 0)),
        ],
        out_specs=pl.BlockSpec((1, 1, N, N), lambda b, hh: (b, SC_H + hh, 0, 0)),
        out_shape=jax.ShapeDtypeStruct((B, H, N, N), jnp.float32),
        scratch_shapes=[pltpu.VMEM((N, T), jnp.float32)],
        input_output_aliases={0: 0},
    )(out_sc, logd, tok, pt_hi)


def kernel(d, tokens, p_table):
    logd, pt = _logd_pass(d, p_table)
    tok = tokens.astype(jnp.int32)
    out_sc = _sc_pass(logd, tok, pt[:SC_H].reshape(SC_H, T * T))
    return _mm_pass(out_sc, logd, tok, pt[SC_H:])


# speedup vs baseline: 1.5947x; 1.5947x over previous
import functools

import jax
import jax.numpy as jnp
import numpy as np
from jax import lax
from jax.experimental import pallas as pl
from jax.experimental.pallas import tpu as pltpu
from jax.experimental.pallas import tpu_sc as plsc

B, N, T, H = 16, 256, 128, 16
LANES = 16
H_PER = 4
CHUNK = 16
NCHUNK = N // CHUNK
B_GRP = B // 8
NITEMS = B_GRP * NCHUNK
NJB = N // LANES

_FMAX = np.float32(np.finfo(np.float32).max)
_FMIN = np.float32(np.finfo(np.float32).min)

_TAKE_DNUMS = lax.GatherDimensionNumbers(
    offset_dims=(), collapsed_slice_dims=(0,), start_index_map=(0,)
)


def _lane_splat(vec, lane):
    idx = jnp.broadcast_to(lane, (LANES,)).astype(jnp.int32)
    return lax.gather(
        vec,
        idx[:, None],
        dimension_numbers=_TAKE_DNUMS,
        slice_sizes=(1,),
        mode=lax.GatherScatterMode.PROMISE_IN_BOUNDS,
    )


def _logd_pass(d, p_table):
    def body(d_ref, pt_ref, o_ref, ptt_ref):
        x = jnp.log(d_ref[...])
        x = jnp.where(jnp.isnan(x), _FMAX, x)
        o_ref[...] = jnp.clip(x, _FMIN, _FMAX)

        @pl.when(pl.program_id(0) == 0)
        def _():
            ptt_ref[...] = jnp.transpose(pt_ref[...], (2, 0, 1))

    return pl.pallas_call(
        body,
        grid=(d.shape[0],),
        in_specs=[
            pl.BlockSpec((1, N, N), lambda b: (b, 0, 0)),
            pl.BlockSpec((T, T, H), lambda b: (0, 0, 0)),
        ],
        out_specs=[
            pl.BlockSpec((1, N, N), lambda b: (b, 0, 0)),
            pl.BlockSpec((H, T, T), lambda b: (0, 0, 0)),
        ],
        out_shape=[
            jax.ShapeDtypeStruct(d.shape, jnp.float32),
            jax.ShapeDtypeStruct((H, T, T), jnp.float32),
        ],
    )(d, p_table)


_mesh = plsc.VectorSubcoreMesh(core_axis_name="c", subcore_axis_name="s")


@functools.partial(
    pl.kernel,
    mesh=_mesh,
    out_type=jax.ShapeDtypeStruct((B, H, N, N), jnp.float32),
    scratch_types=[
        pltpu.VMEM((T * T,), jnp.float32),
        pltpu.VMEM((T * T,), jnp.float32),
        pltpu.VMEM((T * T,), jnp.float32),
        pltpu.VMEM((T * T,), jnp.float32),
        pltpu.VMEM((B_GRP, N), jnp.int32),
        pltpu.VMEM((CHUNK, N), jnp.float32),
        pltpu.VMEM((CHUNK, N), jnp.float32),
        pltpu.VMEM((H_PER, CHUNK, N), jnp.float32),
        pltpu.VMEM((H_PER, CHUNK, N), jnp.float32),
        pltpu.SemaphoreType.DMA,
        pltpu.SemaphoreType.DMA,
        pltpu.SemaphoreType.DMA,
        pltpu.SemaphoreType.DMA,
        pltpu.SemaphoreType.DMA,
    ],
    compiler_params=pltpu.CompilerParams(needs_layout_passes=False),
)
def _sc_pass(logd_hbm, tok_hbm, pt_hbm, out_hbm,
             p_h0, p_h1, p_h2, p_h3, tok_v, ld0, ld1, ob0, ob1,
             is0, is1, os0, os1, psem):
    c = lax.axis_index("c")
    s = lax.axis_index("s")
    h0 = (s % 4) * H_PER
    bg = c * 4 + s // 4
    ld = (ld0, ld1)
    ob = (ob0, ob1)
    isem = (is0, is1)
    osem = (os0, os1)
    planes = (p_h0, p_h1, p_h2, p_h3)

    def item_bcc(g):
        lb = g // NCHUNK
        return lb, bg * B_GRP + lb, g % NCHUNK

    def start_in(g, par):
        _, b, cc = item_bcc(g)
        pltpu.make_async_copy(
            logd_hbm.at[b, pl.ds(cc * CHUNK, CHUNK)], ld[par], isem[par]
        ).start()

    plane_copies = [
        pltpu.make_async_copy(pt_hbm.at[h0 + k], planes[k], psem)
        for k in range(H_PER)
    ]
    tok_copy = pltpu.make_async_copy(
        tok_hbm.at[pl.ds(bg * B_GRP, B_GRP)], tok_v, psem
    )
    for cp in plane_copies:
        cp.start()
    tok_copy.start()
    start_in(0, 0)
    for cp in plane_copies:
        cp.wait()
    tok_copy.wait()

    def pair_body(k, carry):
        for par in (0, 1):
            g = k * 2 + par
            lb, b, cc = item_bcc(g)

            @pl.when(g + 1 < NITEMS)
            def _():
                start_in(g + 1, 1 - par)

            pltpu.make_async_copy(
                logd_hbm.at[b, pl.ds(cc * CHUNK, CHUNK)], ld[par], isem[par]
            ).wait()

            @pl.when(g >= 2)
            def _():
                pltpu.make_async_copy(
                    ob[par],
                    out_hbm.at[b, pl.ds(h0, H_PER), pl.ds(cc * CHUNK, CHUNK)],
                    osem[par],
                ).wait()

            tjs = [tok_v[lb, pl.ds(jb * LANES, LANES)] for jb in range(NJB)]
            ldb = ld[par]
            obb = ob[par]
            tiv = tok_v[lb, pl.ds(cc * CHUNK, CHUNK)]

            @plsc.parallel_loop(0, CHUNK, unroll=2)
            def i_loop(r, tiv=tiv, ldb=ldb, obb=obb, tjs=tjs):
                base = _lane_splat(tiv, r) * T
                for jb in range(NJB):
                    idx = base + tjs[jb]
                    lvec = ldb[r, pl.ds(jb * LANES, LANES)]
                    g0 = plsc.load_gather(p_h0, [idx])
                    g1 = plsc.load_gather(p_h1, [idx])
                    g2 = plsc.load_gather(p_h2, [idx])
                    g3 = plsc.load_gather(p_h3, [idx])
                    obb[0, r, pl.ds(jb * LANES, LANES)] = g0 * lvec
                    obb[1, r, pl.ds(jb * LANES, LANES)] = g1 * lvec
                    obb[2, r, pl.ds(jb * LANES, LANES)] = g2 * lvec
                    obb[3, r, pl.ds(jb * LANES, LANES)] = g3 * lvec

            pltpu.make_async_copy(
                obb,
                out_hbm.at[b, pl.ds(h0, H_PER), pl.ds(cc * CHUNK, CHUNK)],
                osem[par],
            ).start()
        return carry

    lax.fori_loop(0, NITEMS // 2, pair_body, 0)

    for par in (0, 1):
        g = NITEMS - 2 + par
        _, b, cc = item_bcc(g)
        pltpu.make_async_copy(
            ob[par],
            out_hbm.at[b, pl.ds(h0, H_PER), pl.ds(cc * CHUNK, CHUNK)],
            osem[par],
        ).wait()


def kernel(d, tokens, p_table):
    logd, pt = _logd_pass(d, p_table)
    tok = tokens.astype(jnp.int32)
    return _sc_pass(logd, tok, pt.reshape(H, T * T))
